# baseline (device time: 13566 ns/iter reference)
import jax
import jax.numpy as jnp
from jax import lax
from jax.experimental import pallas as pl
from jax.experimental.pallas import tpu as pltpu

N_DEV = 4

A_RELAY, B_RELAY, A_DIRECT, B_DIRECT, A_SUM, B_SUM = range(6)


def kernel(x, w_mat):
    m, _ = x.shape
    _, n = w_mat.shape
    m_out = m // N_DEV
    n2 = n // 2

    def body(x_hbm, w_hbm, out_hbm, xv, wv, outv, sendbuf, recvbuf, f8send,
             f8recv, load_sems, store_sems, send_sems, recv_sems):
        my = lax.axis_index("i")
        left = (my + N_DEV - 1) % N_DEV
        right = (my + 1) % N_DEV

        diag = (my + 2) % N_DEV

        def load_x_chunk(c, k):
            cp = pltpu.make_async_copy(
                x_hbm.at[pl.ds(c * m_out, m_out), :],
                xv.at[pl.ds(c * m_out, m_out), :],
                load_sems.at[k])
            cp.start()
            return cp

        def load_w_half(lo, k):
            cp = pltpu.make_async_copy(
                w_hbm.at[:, pl.ds(lo, n2)], wv.at[:, pl.ds(lo, n2)],
                load_sems.at[k])
            cp.start()
            return cp

        ld_wa = load_w_half(0, 0)
        ld_x0 = load_x_chunk(diag, 1)
        ld_wb = load_w_half(n2, 2)
        ld_x1 = load_x_chunk(right, 3)
        ld_x2 = load_x_chunk(left, 4)
        ld_x3 = load_x_chunk(my, 5)

        barrier_sem = pltpu.get_barrier_semaphore()
        for nbr in (left, right):
            pl.semaphore_signal(
                barrier_sem, inc=1,
                device_id=(nbr,), device_id_type=pl.DeviceIdType.MESH,
            )

        def partial(c, lo, width):
            xb = xv[pl.ds(c * m_out, m_out), :].astype(jnp.bfloat16)
            wb = wv[:, pl.ds(lo, width)].astype(jnp.bfloat16)
            return lax.dot_general(
                xb, wb, (((1,), (0,)), ((), ())),
                preferred_element_type=jnp.float32,
            )

        def copy(k, target):
            return pltpu.make_async_remote_copy(
                src_ref=sendbuf.at[k],
                dst_ref=recvbuf.at[k],
                send_sem=send_sems.at[k],
                recv_sem=recv_sems.at[k],
                device_id=(target,),
                device_id_type=pl.DeviceIdType.MESH,
            )

        def copy_f8(k, target):
            return pltpu.make_async_remote_copy(
                src_ref=f8send.at[k],
                dst_ref=f8recv.at[k],
                send_sem=send_sems.at[k],
                recv_sem=recv_sems.at[k],
                device_id=(target,),
                device_id_type=pl.DeviceIdType.MESH,
            )

        ld_wa.wait()
        ld_x0.wait()
        f8send[A_RELAY, :, :] = partial(diag, 0, n2).astype(jnp.float8_e4m3fn)
        pl.semaphore_wait(barrier_sem, 2)
        r_arelay = copy_f8(A_RELAY, left)
        r_arelay.start()
        ld_wb.wait()
        f8send[B_RELAY, :, :] = partial(diag, n2, n2).astype(jnp.float8_e4m3fn)
        r_brelay = copy_f8(B_RELAY, right)
        r_brelay.start()

        ld_x1.wait()
        p_right = partial(right, 0, n)
        sendbuf[A_DIRECT, :, :] = p_right[:, :n2].astype(jnp.bfloat16)
        r_adirect = copy(A_DIRECT, right)
        r_adirect.start()

        ld_x2.wait()
        p_left = partial(left, 0, n)
        sendbuf[B_DIRECT, :, :] = p_left[:, n2:].astype(jnp.bfloat16)
        r_bdirect = copy(B_DIRECT, left)
        r_bdirect.start()

        ld_x3.wait()
        p_own = partial(my, 0, n)

        copy_f8(A_RELAY, left).wait_recv()
        a_sum = f8recv[A_RELAY, :, :].astype(jnp.float32) + p_left[:, :n2]
        sendbuf[A_SUM, :, :] = a_sum.astype(jnp.bfloat16)
        r_asum = copy(A_SUM, left)
        r_asum.start()

        copy_f8(B_RELAY, right).wait_recv()
        b_sum = f8recv[B_RELAY, :, :].astype(jnp.float32) + p_right[:, n2:]
        sendbuf[B_SUM, :, :] = b_sum.astype(jnp.bfloat16)
        r_bsum = copy(B_SUM, right)
        r_bsum.start()

        copy(A_DIRECT, right).wait_recv()
        pre_a = p_own[:, :n2] + recvbuf[A_DIRECT, :, :].astype(jnp.float32)
        copy(B_DIRECT, left).wait_recv()
        pre_b = p_own[:, n2:] + recvbuf[B_DIRECT, :, :].astype(jnp.float32)

        copy(A_SUM, left).wait_recv()
        outv[:, :n2] = jnp.maximum(
            pre_a + recvbuf[A_SUM, :, :].astype(jnp.float32), 0.0
        ).astype(jnp.bfloat16)
        st_a = pltpu.make_async_copy(
            outv.at[:, pl.ds(0, n2)], out_hbm.at[:, pl.ds(0, n2)],
            store_sems.at[0])
        st_a.start()

        copy(B_SUM, right).wait_recv()
        outv[:, n2:] = jnp.maximum(
            pre_b + recvbuf[B_SUM, :, :].astype(jnp.float32), 0.0
        ).astype(jnp.bfloat16)
        st_b = pltpu.make_async_copy(
            outv.at[:, pl.ds(n2, n2)], out_hbm.at[:, pl.ds(n2, n2)],
            store_sems.at[1])
        st_b.start()

        for r in (r_arelay, r_brelay, r_adirect, r_bdirect, r_asum, r_bsum):
            r.wait_send()
        st_a.wait()
        st_b.wait()

    return pl.pallas_call(
        body,
        out_shape=jax.ShapeDtypeStruct((m_out, n), jnp.bfloat16),
        in_specs=[
            pl.BlockSpec(memory_space=pltpu.MemorySpace.HBM),
            pl.BlockSpec(memory_space=pltpu.MemorySpace.HBM),
        ],
        out_specs=pl.BlockSpec(memory_space=pl.ANY),
        scratch_shapes=[
            pltpu.VMEM((m, x.shape[1]), jnp.float32),
            pltpu.VMEM((m_out, n), jnp.float32),
            pltpu.VMEM((m_out, n), jnp.bfloat16),
            pltpu.VMEM((6, m_out, n2), jnp.bfloat16),
            pltpu.VMEM((6, m_out, n2), jnp.bfloat16),
            pltpu.VMEM((2, m_out, n2), jnp.float8_e4m3fn),
            pltpu.VMEM((2, m_out, n2), jnp.float8_e4m3fn),
            pltpu.SemaphoreType.DMA((6,)),
            pltpu.SemaphoreType.DMA((2,)),
            pltpu.SemaphoreType.DMA((6,)),
            pltpu.SemaphoreType.DMA((6,)),
        ],
        compiler_params=pltpu.CompilerParams(collective_id=0),
    )(
        pltpu.with_memory_space_constraint(x, pltpu.MemorySpace.HBM),
        pltpu.with_memory_space_constraint(w_mat, pltpu.MemorySpace.HBM),
    )


# device time: 13526 ns/iter; 1.0030x vs baseline; 1.0030x over previous
import jax
import jax.numpy as jnp
from jax import lax
from jax.experimental import pallas as pl
from jax.experimental.pallas import tpu as pltpu

N_DEV = 4

A_RELAY, B_RELAY, A_DIRECT, B_DIRECT, A_SUM, B_SUM = range(6)


def kernel(x, w_mat):
    m, _ = x.shape
    _, n = w_mat.shape
    m_out = m // N_DEV
    n2 = n // 2

    def body(x_hbm, w_hbm, out_ref, xv, wv, sendbuf, recvbuf, f8send,
             f8recv, load_sems, send_sems, recv_sems):
        my = lax.axis_index("i")
        left = (my + N_DEV - 1) % N_DEV
        right = (my + 1) % N_DEV

        diag = (my + 2) % N_DEV

        def load_x_chunk(c, k):
            cp = pltpu.make_async_copy(
                x_hbm.at[pl.ds(c * m_out, m_out), :],
                xv.at[pl.ds(c * m_out, m_out), :],
                load_sems.at[k])
            cp.start()
            return cp

        def load_w_half(lo, k):
            cp = pltpu.make_async_copy(
                w_hbm.at[:, pl.ds(lo, n2)], wv.at[:, pl.ds(lo, n2)],
                load_sems.at[k])
            cp.start()
            return cp

        ld_wa = load_w_half(0, 0)
        ld_x0 = load_x_chunk(diag, 1)
        ld_wb = load_w_half(n2, 2)
        ld_x1 = load_x_chunk(right, 3)
        ld_x2 = load_x_chunk(left, 4)
        ld_x3 = load_x_chunk(my, 5)

        barrier_sem = pltpu.get_barrier_semaphore()
        for nbr in (left, right):
            pl.semaphore_signal(
                barrier_sem, inc=1,
                device_id=(nbr,), device_id_type=pl.DeviceIdType.MESH,
            )

        def partial(c, lo, width):
            xb = xv[pl.ds(c * m_out, m_out), :].astype(jnp.bfloat16)
            wb = wv[:, pl.ds(lo, width)].astype(jnp.bfloat16)
            return lax.dot_general(
                xb, wb, (((1,), (0,)), ((), ())),
                preferred_element_type=jnp.float32,
            )

        def copy(k, target):
            return pltpu.make_async_remote_copy(
                src_ref=sendbuf.at[k],
                dst_ref=recvbuf.at[k],
                send_sem=send_sems.at[k],
                recv_sem=recv_sems.at[k],
                device_id=(target,),
                device_id_type=pl.DeviceIdType.MESH,
            )

        def copy_f8(k, target):
            return pltpu.make_async_remote_copy(
                src_ref=f8send.at[k],
                dst_ref=f8recv.at[k],
                send_sem=send_sems.at[k],
                recv_sem=recv_sems.at[k],
                device_id=(target,),
                device_id_type=pl.DeviceIdType.MESH,
            )

        ld_wa.wait()
        ld_x0.wait()
        f8send[A_RELAY, :, :] = partial(diag, 0, n2).astype(jnp.float8_e4m3fn)
        pl.semaphore_wait(barrier_sem, 2)
        r_arelay = copy_f8(A_RELAY, left)
        r_arelay.start()
        ld_wb.wait()
        f8send[B_RELAY, :, :] = partial(diag, n2, n2).astype(jnp.float8_e4m3fn)
        r_brelay = copy_f8(B_RELAY, right)
        r_brelay.start()

        ld_x1.wait()
        p_right = partial(right, 0, n)
        sendbuf[A_DIRECT, :, :] = p_right[:, :n2].astype(jnp.bfloat16)
        r_adirect = copy(A_DIRECT, right)
        r_adirect.start()

        ld_x2.wait()
        p_left = partial(left, 0, n)
        sendbuf[B_DIRECT, :, :] = p_left[:, n2:].astype(jnp.bfloat16)
        r_bdirect = copy(B_DIRECT, left)
        r_bdirect.start()

        ld_x3.wait()
        p_own = partial(my, 0, n)

        copy_f8(A_RELAY, left).wait_recv()
        a_sum = f8recv[A_RELAY, :, :].astype(jnp.float32) + p_left[:, :n2]
        sendbuf[A_SUM, :, :] = a_sum.astype(jnp.bfloat16)
        r_asum = copy(A_SUM, left)
        r_asum.start()

        copy_f8(B_RELAY, right).wait_recv()
        b_sum = f8recv[B_RELAY, :, :].astype(jnp.float32) + p_right[:, n2:]
        sendbuf[B_SUM, :, :] = b_sum.astype(jnp.bfloat16)
        r_bsum = copy(B_SUM, right)
        r_bsum.start()

        copy(A_DIRECT, right).wait_recv()
        pre_a = p_own[:, :n2] + recvbuf[A_DIRECT, :, :].astype(jnp.float32)
        copy(B_DIRECT, left).wait_recv()
        pre_b = p_own[:, n2:] + recvbuf[B_DIRECT, :, :].astype(jnp.float32)

        copy(A_SUM, left).wait_recv()
        out_ref[:, :n2] = jnp.maximum(
            pre_a + recvbuf[A_SUM, :, :].astype(jnp.float32), 0.0
        ).astype(jnp.bfloat16)

        copy(B_SUM, right).wait_recv()
        out_ref[:, n2:] = jnp.maximum(
            pre_b + recvbuf[B_SUM, :, :].astype(jnp.float32), 0.0
        ).astype(jnp.bfloat16)

        for r in (r_arelay, r_brelay, r_adirect, r_bdirect, r_asum, r_bsum):
            r.wait_send()

    return pl.pallas_call(
        body,
        out_shape=jax.ShapeDtypeStruct((m_out, n), jnp.bfloat16),
        in_specs=[
            pl.BlockSpec(memory_space=pltpu.MemorySpace.HBM),
            pl.BlockSpec(memory_space=pltpu.MemorySpace.HBM),
        ],
        out_specs=pl.BlockSpec(memory_space=pltpu.VMEM),
        scratch_shapes=[
            pltpu.VMEM((m, x.shape[1]), jnp.float32),
            pltpu.VMEM((m_out, n), jnp.float32),
            pltpu.VMEM((6, m_out, n2), jnp.bfloat16),
            pltpu.VMEM((6, m_out, n2), jnp.bfloat16),
            pltpu.VMEM((2, m_out, n2), jnp.float8_e4m3fn),
            pltpu.VMEM((2, m_out, n2), jnp.float8_e4m3fn),
            pltpu.SemaphoreType.DMA((6,)),
            pltpu.SemaphoreType.DMA((6,)),
            pltpu.SemaphoreType.DMA((6,)),
        ],
        compiler_params=pltpu.CompilerParams(collective_id=0),
    )(
        pltpu.with_memory_space_constraint(x, pltpu.MemorySpace.HBM),
        pltpu.with_memory_space_constraint(w_mat, pltpu.MemorySpace.HBM),
    )
